# baseline (speedup 1.0000x reference)
import functools

import jax
import jax.numpy as jnp
from jax import lax
from jax.experimental import pallas as pl
from jax.experimental.pallas import tpu as pltpu
from jax.experimental.pallas import tpu_sc as plsc

D_MODEL = 1024
N_TOK = 16384
NC, NS = 2, 16
NW = NC * NS
BPW = N_TOK // NW

CH = 16
K = 3
CAP = BPW + 16


@functools.cache
def _build_sc_gather():
    mesh = plsc.VectorSubcoreMesh(
        core_axis_name="c", subcore_axis_name="s",
        num_cores=NC, num_subcores=NS)

    @functools.partial(
        pl.kernel,
        out_type=(
            jax.ShapeDtypeStruct((N_TOK, D_MODEL), jnp.float32),
            jax.ShapeDtypeStruct((N_TOK, 256), jnp.float32),
            jax.ShapeDtypeStruct((N_TOK, 128), jnp.float32),
        ),
        mesh=mesh,
        compiler_params=pltpu.CompilerParams(needs_layout_passes=False),
        scratch_types=[
            pltpu.VMEM((BPW,), jnp.int32),
            pltpu.VMEM((CAP,), jnp.int32),
            pltpu.VMEM((CAP,), jnp.int32),
            pltpu.VMEM((CAP,), jnp.int32),
            pltpu.VMEM((CAP,), jnp.int32),
            pltpu.VMEM((CAP,), jnp.int32),
            pltpu.VMEM((CAP,), jnp.int32),
            pltpu.VMEM((K, CH, D_MODEL), jnp.float32),
            pltpu.VMEM((K, CH, 256), jnp.float32),
            pltpu.VMEM((K, CH, 128), jnp.float32),
        ] + [pltpu.SemaphoreType.DMA] * (6 * K),
    )
    def _sc_gather(ids_hbm, emb0, emb1, emb2, out0_hbm, g1_hbm, g2_hbm,
                   ids_v, cid0, cpos0, cid1, cpos1, cid2, cpos2,
                   b0, b1, b2, *sems):
        g0s, s0s = sems[0:K], sems[K:2 * K]
        g1s, s1s = sems[2 * K:3 * K], sems[3 * K:4 * K]
        g2s, s2s = sems[4 * K:5 * K], sems[5 * K:6 * K]
        wid = lax.axis_index("s") * NC + lax.axis_index("c")
        base = wid * BPW
        pltpu.sync_copy(ids_hbm.at[pl.ds(base, BPW)], ids_v)

        lane = lax.iota(jnp.int32, 16)

        def compact_one(cid_v, cpos_v, rows, pos, mask, carry):
            cnt, pad_id, pad_pos = carry
            plsc.store_compressed(cid_v.at[pl.ds(cnt, 16)], rows, mask=mask)
            plsc.store_compressed(cpos_v.at[pl.ds(cnt, 16)], pos, mask=mask)
            npop = jnp.max(plsc.all_reduce_population_count(mask))
            mpos = jnp.max(jnp.where(mask, pos, -1))
            mid = jnp.max(
                jnp.where(jnp.where(mask, pos, -1) == mpos, rows, -1))
            has = mpos >= 0
            return (cnt + npop,
                    jnp.where(has, mid, pad_id),
                    jnp.where(has, mpos, pad_pos))

        def compute_idx(i, carry):
            c0c, c1c, c2c = carry
            v = ids_v[pl.ds(i * 16, 16)]
            pos = base + i * 16 + lane
            m0 = v < 20000
            m1 = (v >= 20000) & (v < 60000)
            m2 = v >= 60000
            c0c = compact_one(cid0, cpos0, v, pos, m0, c0c)
            c1c = compact_one(cid1, cpos1, v - 20000, pos, m1, c1c)
            c2c = compact_one(
                cid2, cpos2, jnp.right_shift(v - 60000, 1), pos, m2, c2c)
            return (c0c, c1c, c2c)

        zero = (0, 0, 0)
        (cnt0, pid0, ppos0), (cnt1, pid1, ppos1), (cnt2, pid2, ppos2) = (
            lax.fori_loop(0, BPW // 16, compute_idx, (zero, zero, zero)))
        cid0[pl.ds(cnt0, 16)] = jnp.full((16,), pid0, jnp.int32)
        cpos0[pl.ds(cnt0, 16)] = jnp.full((16,), ppos0, jnp.int32)
        cid1[pl.ds(cnt1, 16)] = jnp.full((16,), pid1, jnp.int32)
        cpos1[pl.ds(cnt1, 16)] = jnp.full((16,), ppos1, jnp.int32)
        cid2[pl.ds(cnt2, 16)] = jnp.full((16,), pid2, jnp.int32)
        cpos2[pl.ds(cnt2, 16)] = jnp.full((16,), ppos2, jnp.int32)
        nch0 = jnp.right_shift(cnt0 + CH - 1, 4)
        nch1 = jnp.right_shift(cnt1 + CH - 1, 4)
        nch2 = jnp.right_shift(cnt2 + CH - 1, 4)

        def make_chain(table, out_hbm, cid_v, cpos_v, buf, gsems, ssems):
            def fire_g(c, slot):
                iv = cid_v[pl.ds(c * CH, CH)]
                pltpu.async_copy(table.at[iv], buf.at[slot], gsems[slot])

            def wait_g(c, slot):
                iv = cid_v[pl.ds(c * CH, CH)]
                pltpu.make_async_copy(
                    table.at[iv], buf.at[slot], gsems[slot]).wait()

            def fire_s(c, slot):
                pv = cpos_v[pl.ds(c * CH, CH)]
                pltpu.async_copy(buf.at[slot], out_hbm.at[pv], ssems[slot])

            def wait_s(c, slot):
                pv = cpos_v[pl.ds(c * CH, CH)]
                pltpu.make_async_copy(
                    buf.at[slot], out_hbm.at[pv], ssems[slot]).wait()

            return fire_g, wait_g, fire_s, wait_s

        chains = (
            make_chain(emb0, out0_hbm, cid0, cpos0, b0, g0s, s0s) + (nch0,),
            make_chain(emb1, g1_hbm, cid1, cpos1, b1, g1s, s1s) + (nch1,),
            make_chain(emb2, g2_hbm, cid2, cpos2, b2, g2s, s2s) + (nch2,),
        )

        for i in range(K):
            for fire_g, _, _, _, nch in chains:
                @pl.when(nch > i)
                def _pro(fire_g=fire_g, i=i):
                    fire_g(i, i)

        nmax = jnp.maximum(jnp.maximum(nch0, nch1), nch2)

        def step(c, carry):
            for fire_g, wait_g, fire_s, wait_s, nch in chains:
                for s in range(K):
                    ns = (s + 1) % K

                    @pl.when((nch > c) & (lax.rem(c, jnp.int32(K)) == s))
                    def _adv(fire_g=fire_g, wait_g=wait_g, fire_s=fire_s,
                             wait_s=wait_s, nch=nch, s=s, ns=ns):
                        wait_g(c, s)
                        fire_s(c, s)

                        @pl.when((c >= K - 1) & (nch > c + 1))
                        def _refire():
                            wait_s(c - (K - 1), ns)
                            fire_g(c + 1, ns)
            return carry

        lax.fori_loop(0, nmax, step, 0)

        for _, _, _, wait_s, nch in chains:
            for j in range(1, K + 1):
                @pl.when(nch >= j)
                def _tail(wait_s=wait_s, nch=nch, j=j):
                    c = nch - j
                    for s in range(K):
                        @pl.when(lax.rem(c, jnp.int32(K)) == s)
                        def _w(wait_s=wait_s, s=s, c=c):
                            wait_s(c, s)

    return _sc_gather


BT = 2048


def _tc_body(ids_ref, p0_ref, g1_ref, g2_ref, w1_ref, w2_ref, out_ref):
    ids = ids_ref[...]
    m1 = (ids >= 20000) & (ids < 60000)
    m2 = ids >= 60000
    g1 = jnp.where(m1, g1_ref[...], 0.0)
    g2pair = g2_ref[...]
    odd = ((ids - 60000) & 1) == 1
    g2 = jnp.where(m2 & odd, g2pair[:, 64:],
                   jnp.where(m2, g2pair[:, :64], 0.0))
    acc = jnp.dot(g1, w1_ref[...], preferred_element_type=jnp.float32)
    acc = acc + jnp.dot(g2, w2_ref[...], preferred_element_type=jnp.float32)
    out_ref[...] = jnp.where(ids < 20000, p0_ref[...], acc)


_tc_combine = pl.pallas_call(
    _tc_body,
    grid=(N_TOK // BT,),
    in_specs=[
        pl.BlockSpec((BT, 1), lambda i: (i, 0)),
        pl.BlockSpec((BT, D_MODEL), lambda i: (i, 0)),
        pl.BlockSpec((BT, 256), lambda i: (i, 0)),
        pl.BlockSpec((BT, 128), lambda i: (i, 0)),
        pl.BlockSpec((256, D_MODEL), lambda i: (0, 0)),
        pl.BlockSpec((64, D_MODEL), lambda i: (0, 0)),
    ],
    out_specs=pl.BlockSpec((BT, D_MODEL), lambda i: (i, 0)),
    out_shape=jax.ShapeDtypeStruct((N_TOK, D_MODEL), jnp.float32),
    input_output_aliases={1: 0},
)


def kernel(input_ids, emb0, emb1, emb2, W1, W2):
    ids = input_ids.reshape(-1).astype(jnp.int32)
    emb2r = emb2.reshape(20000, 128)
    out0, g1, g2 = _build_sc_gather()(ids, emb0, emb1, emb2r)
    out = _tc_combine(ids.reshape(N_TOK, 1), out0, g1, g2, W1, W2)
    return out.reshape(input_ids.shape + (D_MODEL,))


# speedup vs baseline: 1.0103x; 1.0103x over previous
import functools

import jax
import jax.numpy as jnp
from jax import lax
from jax.experimental import pallas as pl
from jax.experimental.pallas import tpu as pltpu
from jax.experimental.pallas import tpu_sc as plsc

D_MODEL = 1024
N_TOK = 16384
NC, NS = 2, 16
NW = NC * NS
BPW = N_TOK // NW

CH = 16
K = 3
CAP = BPW + 16


@functools.cache
def _build_sc_gather():
    mesh = plsc.VectorSubcoreMesh(
        core_axis_name="c", subcore_axis_name="s",
        num_cores=NC, num_subcores=NS)

    @functools.partial(
        pl.kernel,
        out_type=(
            jax.ShapeDtypeStruct((N_TOK, D_MODEL), jnp.float32),
            jax.ShapeDtypeStruct((N_TOK, 256), jnp.float32),
            jax.ShapeDtypeStruct((N_TOK, 128), jnp.float32),
        ),
        mesh=mesh,
        compiler_params=pltpu.CompilerParams(needs_layout_passes=False),
        scratch_types=[
            pltpu.VMEM((BPW,), jnp.int32),
            pltpu.VMEM((CAP,), jnp.int32),
            pltpu.VMEM((CAP,), jnp.int32),
            pltpu.VMEM((CAP,), jnp.int32),
            pltpu.VMEM((CAP,), jnp.int32),
            pltpu.VMEM((CAP,), jnp.int32),
            pltpu.VMEM((CAP,), jnp.int32),
            pltpu.VMEM((K, CH, D_MODEL), jnp.float32),
            pltpu.VMEM((K, CH, 256), jnp.float32),
            pltpu.VMEM((K, CH, 128), jnp.float32),
        ] + [pltpu.SemaphoreType.DMA] * (6 * K),
    )
    def _sc_gather(ids_hbm, emb0, emb1, emb2, out0_hbm, g1_hbm, g2_hbm,
                   ids_v, cid0, cpos0, cid1, cpos1, cid2, cpos2,
                   b0, b1, b2, *sems):
        g0s, s0s = sems[0:K], sems[K:2 * K]
        g1s, s1s = sems[2 * K:3 * K], sems[3 * K:4 * K]
        g2s, s2s = sems[4 * K:5 * K], sems[5 * K:6 * K]
        wid = lax.axis_index("s") * NC + lax.axis_index("c")
        base = wid * BPW
        pltpu.sync_copy(ids_hbm.at[pl.ds(base, BPW)], ids_v)

        lane = lax.iota(jnp.int32, 16)

        def compact_one(cid_v, cpos_v, rows, pos, mask, carry):
            cnt, pad_id, pad_pos = carry
            plsc.store_compressed(cid_v.at[pl.ds(cnt, 16)], rows, mask=mask)
            plsc.store_compressed(cpos_v.at[pl.ds(cnt, 16)], pos, mask=mask)
            npop = jnp.max(plsc.all_reduce_population_count(mask))
            mpos = jnp.max(jnp.where(mask, pos, -1))
            mid = jnp.max(
                jnp.where(jnp.where(mask, pos, -1) == mpos, rows, -1))
            has = mpos >= 0
            return (cnt + npop,
                    jnp.where(has, mid, pad_id),
                    jnp.where(has, mpos, pad_pos))

        def compute_idx(i, carry):
            c0c, c1c, c2c = carry
            v = ids_v[pl.ds(i * 16, 16)]
            pos = base + i * 16 + lane
            m0 = v < 20000
            m1 = (v >= 20000) & (v < 60000)
            m2 = v >= 60000
            c0c = compact_one(cid0, cpos0, v, pos, m0, c0c)
            c1c = compact_one(cid1, cpos1, v - 20000, pos, m1, c1c)
            c2c = compact_one(
                cid2, cpos2, jnp.right_shift(v - 60000, 1), pos, m2, c2c)
            return (c0c, c1c, c2c)

        zero = (0, 0, 0)
        (cnt0, pid0, ppos0), (cnt1, pid1, ppos1), (cnt2, pid2, ppos2) = (
            lax.fori_loop(0, BPW // 16, compute_idx, (zero, zero, zero)))
        cid0[pl.ds(cnt0, 16)] = jnp.full((16,), pid0, jnp.int32)
        cpos0[pl.ds(cnt0, 16)] = jnp.full((16,), ppos0, jnp.int32)
        cid1[pl.ds(cnt1, 16)] = jnp.full((16,), pid1, jnp.int32)
        cpos1[pl.ds(cnt1, 16)] = jnp.full((16,), ppos1, jnp.int32)
        cid2[pl.ds(cnt2, 16)] = jnp.full((16,), pid2, jnp.int32)
        cpos2[pl.ds(cnt2, 16)] = jnp.full((16,), ppos2, jnp.int32)
        nch0 = jnp.right_shift(cnt0 + CH - 1, 4)
        nch1 = jnp.right_shift(cnt1 + CH - 1, 4)
        nch2 = jnp.right_shift(cnt2 + CH - 1, 4)

        def make_chain(table, out_hbm, cid_v, cpos_v, buf, gsems, ssems):
            def fire_g(c, slot):
                iv = cid_v[pl.ds(c * CH, CH)]
                pltpu.async_copy(table.at[iv], buf.at[slot], gsems[slot])

            def wait_g(c, slot):
                iv = cid_v[pl.ds(c * CH, CH)]
                pltpu.make_async_copy(
                    table.at[iv], buf.at[slot], gsems[slot]).wait()

            def fire_s(c, slot):
                pv = cpos_v[pl.ds(c * CH, CH)]
                pltpu.async_copy(buf.at[slot], out_hbm.at[pv], ssems[slot])

            def wait_s(c, slot):
                pv = cpos_v[pl.ds(c * CH, CH)]
                pltpu.make_async_copy(
                    buf.at[slot], out_hbm.at[pv], ssems[slot]).wait()

            return fire_g, wait_g, fire_s, wait_s

        chains = (
            make_chain(emb0, out0_hbm, cid0, cpos0, b0, g0s, s0s) + (nch0,),
            make_chain(emb1, g1_hbm, cid1, cpos1, b1, g1s, s1s) + (nch1,),
            make_chain(emb2, g2_hbm, cid2, cpos2, b2, g2s, s2s) + (nch2,),
        )

        for i in range(K):
            for fire_g, _, _, _, nch in chains:
                @pl.when(nch > i)
                def _pro(fire_g=fire_g, i=i):
                    fire_g(i, i)

        nmax = jnp.maximum(jnp.maximum(nch0, nch1), nch2)

        def step(c, carry):
            for fire_g, wait_g, fire_s, wait_s, nch in chains:
                for s in range(K):
                    ns = (s + 1) % K

                    @pl.when((nch > c) & (lax.rem(c, jnp.int32(K)) == s))
                    def _adv(fire_g=fire_g, wait_g=wait_g, fire_s=fire_s,
                             wait_s=wait_s, nch=nch, s=s, ns=ns):
                        wait_g(c, s)
                        fire_s(c, s)

                        @pl.when((c >= K - 1) & (nch > c + 1))
                        def _refire():
                            wait_s(c - (K - 1), ns)
                            fire_g(c + 1, ns)
            return carry

        lax.fori_loop(0, nmax, step, 0)

        for _, _, _, wait_s, nch in chains:
            for j in range(1, K + 1):
                @pl.when(nch >= j)
                def _tail(wait_s=wait_s, nch=nch, j=j):
                    c = nch - j
                    for s in range(K):
                        @pl.when(lax.rem(c, jnp.int32(K)) == s)
                        def _w(wait_s=wait_s, s=s, c=c):
                            wait_s(c, s)

    return _sc_gather


BT = 2048


def _tc_body(ids_ref, p0_ref, g1_ref, g2_ref, w1_ref, w2_ref, out_ref):
    ids = ids_ref[...]
    m1 = (ids >= 20000) & (ids < 60000)
    m2 = ids >= 60000
    g1 = jnp.where(m1, g1_ref[...], 0.0)
    lane = lax.broadcasted_iota(jnp.int32, (BT, 128), 1)
    parity = (ids - 60000) & 1
    half_ok = (lane >= 64) == (parity == 1)
    g2 = jnp.where(m2 & half_ok, g2_ref[...], 0.0)
    w2 = w2_ref[...]
    w2x = jnp.concatenate([w2, w2], axis=0)
    acc = jnp.dot(g1, w1_ref[...], preferred_element_type=jnp.float32)
    acc = acc + jnp.dot(g2, w2x, preferred_element_type=jnp.float32)
    out_ref[...] = jnp.where(ids < 20000, p0_ref[...], acc)


_tc_combine = pl.pallas_call(
    _tc_body,
    grid=(N_TOK // BT,),
    in_specs=[
        pl.BlockSpec((BT, 1), lambda i: (i, 0)),
        pl.BlockSpec((BT, D_MODEL), lambda i: (i, 0)),
        pl.BlockSpec((BT, 256), lambda i: (i, 0)),
        pl.BlockSpec((BT, 128), lambda i: (i, 0)),
        pl.BlockSpec((256, D_MODEL), lambda i: (0, 0)),
        pl.BlockSpec((64, D_MODEL), lambda i: (0, 0)),
    ],
    out_specs=pl.BlockSpec((BT, D_MODEL), lambda i: (i, 0)),
    out_shape=jax.ShapeDtypeStruct((N_TOK, D_MODEL), jnp.float32),
    input_output_aliases={1: 0},
)


def kernel(input_ids, emb0, emb1, emb2, W1, W2):
    ids = input_ids.reshape(-1).astype(jnp.int32)
    emb2r = emb2.reshape(20000, 128)
    out0, g1, g2 = _build_sc_gather()(ids, emb0, emb1, emb2r)
    out = _tc_combine(ids.reshape(N_TOK, 1), out0, g1, g2, W1, W2)
    return out.reshape(input_ids.shape + (D_MODEL,))
